# baseline (speedup 1.0000x reference)
import functools

import jax
import jax.numpy as jnp
from jax import lax
from jax.experimental import pallas as pl
from jax.experimental.pallas import tpu as pltpu, tpu_sc as plsc

L = 16
NS = 16


def _sc_graph_step(vals_pad, src, dst, w, *, npad, e):
    ept = e // NS
    chunk = 8000
    assert ept % chunk == 0
    nchunks = ept // chunk
    slc = npad // NS
    assert slc % L == 0

    mesh = plsc.VectorSubcoreMesh(
        core_axis_name="c", subcore_axis_name="s", num_cores=1)

    @functools.partial(
        pl.kernel,
        out_type=[
            jax.ShapeDtypeStruct((npad * 3,), jnp.float32),
            jax.ShapeDtypeStruct((npad,), jnp.float32),
            jax.ShapeDtypeStruct((NS * npad,), jnp.float32),
        ],
        mesh=mesh,
        compiler_params=pltpu.CompilerParams(needs_layout_passes=False),
        scratch_types=[
            pltpu.VMEM((npad,), jnp.float32),
            pltpu.VMEM((npad,), jnp.float32),
            pltpu.VMEM((chunk,), jnp.int32),
            pltpu.VMEM((chunk,), jnp.int32),
            pltpu.VMEM((chunk,), jnp.float32),
            pltpu.VMEM((chunk,), jnp.int32),
            pltpu.VMEM((chunk,), jnp.int32),
            pltpu.VMEM((chunk,), jnp.float32),
            pltpu.SemaphoreType.DMA,
            pltpu.SemaphoreType.DMA,
            pltpu.SemaphoreType.DMA,
            pltpu.VMEM((NS * slc,), jnp.float32),
            pltpu.VMEM((slc * 3,), jnp.float32),
            pltpu.VMEM((slc,), jnp.float32),
            pltpu.VMEM((slc,), jnp.float32),
            pltpu.VMEM((slc,), jnp.float32),
            pltpu.VMEM((slc,), jnp.float32),
        ],
    )
    def body(x_hbm, src_hbm, dst_hbm, w_hbm,
             out_hbm, tab_hbm, part_hbm,
             tab_v, acc_v, src_v0, dst_v0, w_v0, src_v1, dst_v1, w_v1,
             esem0, esem1, rsem, red_v, x3_v,
             vals_s, fx_s, err_s, sum_s):
        ebufs = ((src_v0, dst_v0, w_v0), (src_v1, dst_v1, w_v1))
        esems = (esem0, esem1)
        s = lax.axis_index("s")
        base = s * slc
        ebase = s * ept

        def start_chunk(c):
            b = c % 2
            eoff = ebase + c * chunk
            return [pltpu.async_copy(src_hbm.at[pl.ds(eoff, chunk)],
                                     ebufs[b][0], esems[b]),
                    pltpu.async_copy(dst_hbm.at[pl.ds(eoff, chunk)],
                                     ebufs[b][1], esems[b]),
                    pltpu.async_copy(w_hbm.at[pl.ds(eoff, chunk)],
                                     ebufs[b][2], esems[b])]

        pending = {0: start_chunk(0), 1: start_chunk(1)}

        pltpu.sync_copy(x_hbm.at[pl.ds(base * 3, slc * 3)], x3_v)
        lanes = lax.iota(jnp.int32, L)
        for i in range(slc // L):
            v = plsc.load_gather(x3_v, [(lanes + i * L) * 3])
            vals_s[pl.ds(i * L, L)] = v
            fx_s[pl.ds(i * L, L)] = 1.0 - 2.0 / (jnp.exp(2.0 * v) + 1.0)
        pltpu.sync_copy(fx_s, tab_hbm.at[pl.ds(base, slc)])
        plsc.subcore_barrier()
        pltpu.sync_copy(tab_hbm, tab_v)

        def zero_acc():
            @pl.loop(0, npad, step=L, unroll=8)
            def _(i):
                acc_v[pl.ds(i, L)] = jnp.zeros((L,), jnp.float32)

        def edge_pass(gather_first):
            for c in range(nchunks):
                for cp in pending.pop(c):
                    cp.wait()
                src_v, dst_v, w_v = ebufs[c % 2]

                @plsc.parallel_loop(0, chunk, L, unroll=8)
                def _(i):
                    sv = src_v[pl.ds(i, L)]
                    dv = dst_v[pl.ds(i, L)]
                    wv = w_v[pl.ds(i, L)]
                    gidx = sv if gather_first else dv
                    sidx = dv if gather_first else sv
                    g = plsc.load_gather(tab_v, [gidx])
                    plsc.addupdate_scatter(acc_v, [sidx], wv * g)

                if c + 2 <= nchunks - 1:
                    pending[c + 2] = start_chunk(c + 2)

        def reduce_partials(out_s):
            pltpu.sync_copy(acc_v, part_hbm.at[pl.ds(s * npad, npad)])
            plsc.subcore_barrier()
            cps = [pltpu.async_copy(part_hbm.at[pl.ds(j * npad + base, slc)],
                                    red_v.at[pl.ds(j * slc, slc)], rsem)
                   for j in range(NS)]
            for cp in cps:
                cp.wait()

            @pl.loop(0, slc, step=L, unroll=2)
            def _(i):
                t = red_v[pl.ds(i, L)]
                for j in range(1, NS):
                    t = t + red_v[pl.ds(j * slc + i, L)]
                out_s[pl.ds(i, L)] = t

        zero_acc()
        edge_pass(gather_first=True)
        pending[0] = start_chunk(0)
        pending[1] = start_chunk(1)
        reduce_partials(sum_s)

        for i in range(slc // L):
            err_s[pl.ds(i * L, L)] = vals_s[pl.ds(i * L, L)] - sum_s[pl.ds(i * L, L)]
        pltpu.sync_copy(err_s, tab_hbm.at[pl.ds(base, slc)])
        for i in range(slc // L):
            idx3 = (lanes + i * L) * 3
            plsc.store_scatter(x3_v, [idx3], sum_s[pl.ds(i * L, L)])
            plsc.store_scatter(x3_v, [idx3 + 1], err_s[pl.ds(i * L, L)])
        plsc.subcore_barrier()
        pltpu.sync_copy(tab_hbm, tab_v)

        zero_acc()
        edge_pass(gather_first=False)
        reduce_partials(sum_s)

        for i in range(slc // L):
            fx = fx_s[pl.ds(i * L, L)]
            d = (1.0 - fx * fx) * sum_s[pl.ds(i * L, L)] - err_s[pl.ds(i * L, L)]
            plsc.store_scatter(x3_v, [(lanes + i * L) * 3 + 2], d)
        pltpu.sync_copy(x3_v, out_hbm.at[pl.ds(base * 3, slc * 3)])

    return body(vals_pad, src, dst, w)


def kernel(x, edge_index, weights):
    n = x.shape[0]
    e = edge_index.shape[1]
    npad = ((n + NS * L - 1) // (NS * L)) * (NS * L)
    x_pad = jnp.pad(x, ((0, npad - n), (0, 0))).reshape(-1)
    out_flat, _, _ = _sc_graph_step(
        x_pad, edge_index[0], edge_index[1], weights, npad=npad, e=e)
    return out_flat.reshape(npad, 3)[:n]


# speedup vs baseline: 1.2537x; 1.2537x over previous
import functools

import jax
import jax.numpy as jnp
from jax import lax
from jax.experimental import pallas as pl
from jax.experimental.pallas import tpu as pltpu, tpu_sc as plsc

L = 16
NS = 16


def _sc_graph_step(vals_pad, src, dst, w, *, npad, e):
    ept = e // NS
    chunk = 8000
    assert ept % chunk == 0
    nchunks = ept // chunk
    slc = npad // NS
    assert slc % L == 0

    mesh = plsc.VectorSubcoreMesh(
        core_axis_name="c", subcore_axis_name="s", num_cores=1)

    @functools.partial(
        pl.kernel,
        out_type=[
            jax.ShapeDtypeStruct((npad,), jnp.float32),
            jax.ShapeDtypeStruct((npad,), jnp.float32),
            jax.ShapeDtypeStruct((npad,), jnp.float32),
            jax.ShapeDtypeStruct((npad,), jnp.float32),
            jax.ShapeDtypeStruct((NS * npad,), jnp.float32),
        ],
        mesh=mesh,
        compiler_params=pltpu.CompilerParams(needs_layout_passes=False),
        scratch_types=[
            pltpu.VMEM((npad,), jnp.float32),
            pltpu.VMEM((npad,), jnp.float32),
            pltpu.VMEM((chunk,), jnp.int32),
            pltpu.VMEM((chunk,), jnp.int32),
            pltpu.VMEM((chunk,), jnp.float32),
            pltpu.VMEM((chunk,), jnp.int32),
            pltpu.VMEM((chunk,), jnp.int32),
            pltpu.VMEM((chunk,), jnp.float32),
            pltpu.SemaphoreType.DMA,
            pltpu.SemaphoreType.DMA,
            pltpu.SemaphoreType.DMA,
            pltpu.VMEM((NS * slc,), jnp.float32),
            pltpu.VMEM((slc,), jnp.float32),
            pltpu.VMEM((slc,), jnp.float32),
            pltpu.VMEM((slc,), jnp.float32),
            pltpu.VMEM((slc,), jnp.float32),
        ],
    )
    def body(vals_hbm, src_hbm, dst_hbm, w_hbm,
             preds_hbm, err_hbm, delta_hbm, tab_hbm, part_hbm,
             tab_v, acc_v, src_v0, dst_v0, w_v0, src_v1, dst_v1, w_v1,
             esem0, esem1, rsem, red_v,
             vals_s, fx_s, err_s, sum_s):
        ebufs = ((src_v0, dst_v0, w_v0), (src_v1, dst_v1, w_v1))
        esems = (esem0, esem1)
        s = lax.axis_index("s")
        base = s * slc
        ebase = s * ept

        def start_chunk(c):
            b = c % 2
            eoff = ebase + c * chunk
            return [pltpu.async_copy(src_hbm.at[pl.ds(eoff, chunk)],
                                     ebufs[b][0], esems[b]),
                    pltpu.async_copy(dst_hbm.at[pl.ds(eoff, chunk)],
                                     ebufs[b][1], esems[b]),
                    pltpu.async_copy(w_hbm.at[pl.ds(eoff, chunk)],
                                     ebufs[b][2], esems[b])]

        pending = {0: start_chunk(0), 1: start_chunk(1)}

        pltpu.sync_copy(vals_hbm.at[pl.ds(base, slc)], vals_s)
        for i in range(slc // L):
            v = vals_s[pl.ds(i * L, L)]
            fx_s[pl.ds(i * L, L)] = 1.0 - 2.0 / (jnp.exp(2.0 * v) + 1.0)
        pltpu.sync_copy(fx_s, tab_hbm.at[pl.ds(base, slc)])
        plsc.subcore_barrier()
        pltpu.sync_copy(tab_hbm, tab_v)

        def zero_acc():
            @pl.loop(0, npad, step=L, unroll=8)
            def _(i):
                acc_v[pl.ds(i, L)] = jnp.zeros((L,), jnp.float32)

        def edge_pass(gather_first):
            for c in range(nchunks):
                for cp in pending.pop(c):
                    cp.wait()
                src_v, dst_v, w_v = ebufs[c % 2]

                @plsc.parallel_loop(0, chunk, L, unroll=8)
                def _(i):
                    sv = src_v[pl.ds(i, L)]
                    dv = dst_v[pl.ds(i, L)]
                    wv = w_v[pl.ds(i, L)]
                    gidx = sv if gather_first else dv
                    sidx = dv if gather_first else sv
                    g = plsc.load_gather(tab_v, [gidx])
                    plsc.addupdate_scatter(acc_v, [sidx], wv * g)

                if c + 2 <= nchunks - 1:
                    pending[c + 2] = start_chunk(c + 2)

        def reduce_partials(out_s):
            pltpu.sync_copy(acc_v, part_hbm.at[pl.ds(s * npad, npad)])
            plsc.subcore_barrier()
            cps = [pltpu.async_copy(part_hbm.at[pl.ds(j * npad + base, slc)],
                                    red_v.at[pl.ds(j * slc, slc)], rsem)
                   for j in range(NS)]
            zero_acc()
            for cp in cps:
                cp.wait()

            @pl.loop(0, slc, step=L, unroll=2)
            def _(i):
                t = red_v[pl.ds(i, L)]
                for j in range(1, NS):
                    t = t + red_v[pl.ds(j * slc + i, L)]
                out_s[pl.ds(i, L)] = t

        zero_acc()
        edge_pass(gather_first=True)
        pending[0] = start_chunk(0)
        pending[1] = start_chunk(1)
        reduce_partials(sum_s)

        for i in range(slc // L):
            err_s[pl.ds(i * L, L)] = vals_s[pl.ds(i * L, L)] - sum_s[pl.ds(i * L, L)]
        pltpu.sync_copy(err_s, err_hbm.at[pl.ds(base, slc)])
        pltpu.sync_copy(sum_s, preds_hbm.at[pl.ds(base, slc)])
        plsc.subcore_barrier()
        pltpu.sync_copy(err_hbm, tab_v)

        edge_pass(gather_first=False)
        reduce_partials(sum_s)

        for i in range(slc // L):
            fx = fx_s[pl.ds(i * L, L)]
            err_s[pl.ds(i * L, L)] = (1.0 - fx * fx) * sum_s[pl.ds(i * L, L)] - err_s[pl.ds(i * L, L)]
        pltpu.sync_copy(err_s, delta_hbm.at[pl.ds(base, slc)])

    return body(vals_pad, src, dst, w)


def kernel(x, edge_index, weights):
    n = x.shape[0]
    e = edge_index.shape[1]
    npad = ((n + NS * L - 1) // (NS * L)) * (NS * L)
    vals = x[:, 0]
    vals_pad = jnp.zeros((npad,), jnp.float32).at[:n].set(vals)
    preds, errors, delta, _, _ = _sc_graph_step(
        vals_pad, edge_index[0], edge_index[1], weights, npad=npad, e=e)
    return jnp.stack([preds[:n], errors[:n], delta[:n]], axis=1)
